# baseline (speedup 1.0000x reference)
import jax
import jax.numpy as jnp
from jax.experimental import pallas as pl
from jax.experimental.pallas import tpu as pltpu
from jax.experimental.pallas import tpu_sc as plsc

_BLOCK_ROWS = 16
_LANES = 16


def kernel(x, rate_id, rate_embedding):
    b, s, d = x.shape
    rows = b * s
    x2 = x.reshape(rows, d)
    idx = jnp.asarray([rate_id], dtype=jnp.int32)

    def sc_body(idx_hbm, x_hbm, emb_hbm, o_hbm, rowbuf, idx_vmem, sem):
        pltpu.async_copy(idx_hbm, idx_vmem, sem).wait()
        pltpu.sync_copy(emb_hbm.at[idx_vmem], rowbuf)

        def add_block(x_vmem, o_vmem):
            @pl.loop(0, _BLOCK_ROWS)
            def _(r):
                @pl.loop(0, d, step=_LANES)
                def _(c):
                    o_vmem[r, pl.ds(c, _LANES)] = (
                        x_vmem[r, pl.ds(c, _LANES)] + rowbuf[0, pl.ds(c, _LANES)]
                    )

        pltpu.emit_pipeline(
            add_block,
            grid=(rows // _BLOCK_ROWS,),
            in_specs=[pl.BlockSpec((_BLOCK_ROWS, d), lambda i: (i, 0))],
            out_specs=[pl.BlockSpec((_BLOCK_ROWS, d), lambda i: (i, 0))],
            core_axis_name=("c", "s"),
            dimension_semantics=(pltpu.PARALLEL,),
        )(x_hbm, o_hbm)

    out = pl.kernel(
        sc_body,
        out_type=jax.ShapeDtypeStruct((rows, d), x.dtype),
        mesh=plsc.VectorSubcoreMesh(core_axis_name="c", subcore_axis_name="s"),
        scratch_types=[
            pltpu.VMEM((1, d), x.dtype),
            pltpu.VMEM((1,), jnp.int32),
            pltpu.SemaphoreType.DMA,
        ],
    )(idx, x2, rate_embedding)
    return out.reshape(b, s, d)


# speedup vs baseline: 5.4810x; 5.4810x over previous
import jax
import jax.numpy as jnp
from jax.experimental import pallas as pl
from jax.experimental.pallas import tpu as pltpu

_BLOCK_ROWS = 2048


def _add_row_kernel(idx_ref, emb_ref, x_ref, o_ref):
    row = emb_ref[idx_ref[0], :]
    o_ref[...] = x_ref[...] + row[None, :]


def kernel(x, rate_id, rate_embedding):
    b, s, d = x.shape
    rows = b * s
    x2 = x.reshape(rows, d)
    block = min(_BLOCK_ROWS, rows)
    idx = jnp.asarray([rate_id], dtype=jnp.int32)
    out = pl.pallas_call(
        _add_row_kernel,
        grid_spec=pltpu.PrefetchScalarGridSpec(
            num_scalar_prefetch=1,
            grid=(rows // block,),
            in_specs=[
                pl.BlockSpec(rate_embedding.shape, lambda i, idx_ref: (0, 0)),
                pl.BlockSpec((block, d), lambda i, idx_ref: (i, 0)),
            ],
            out_specs=pl.BlockSpec((block, d), lambda i, idx_ref: (i, 0)),
        ),
        out_shape=jax.ShapeDtypeStruct((rows, d), x.dtype),
        compiler_params=pltpu.CompilerParams(
            dimension_semantics=("arbitrary",),
        ),
    )(idx, rate_embedding, x2)
    return out.reshape(b, s, d)
